# baseline (speedup 1.0000x reference)
import jax
import jax.numpy as jnp
from jax import lax
from jax.experimental import pallas as pl
from jax.experimental.pallas import tpu as pltpu

BLANK_ID = 0
NUM_CLASSES = 1000
BATCH = 1024
TIME = 50
BB = 16
STEPS = BATCH // BB
NBUF = 8


def _one_hot_body(ids_ref, out_ref, buf, sems):
    i = pl.program_id(0)
    slot = lax.rem(i, NBUF)

    @pl.when(i >= NBUF)
    def _wait_reuse():
        pltpu.make_async_copy(
            buf.at[slot],
            out_ref.at[pl.ds((i - NBUF) * BB, BB)],
            sems.at[slot],
        ).wait()

    ids = ids_ref[...]
    sel = jnp.where(ids == BLANK_ID, -1, ids)
    iota = lax.broadcasted_iota(jnp.int32, (BB, TIME, NUM_CLASSES), 2)
    buf[slot] = (iota == sel).astype(jnp.float32)

    pltpu.make_async_copy(
        buf.at[slot], out_ref.at[pl.ds(i * BB, BB)], sems.at[slot]
    ).start()

    @pl.when(i == STEPS - 1)
    def _drain():
        for s in range(NBUF):
            pltpu.make_async_copy(
                buf.at[s],
                out_ref.at[pl.ds((STEPS - NBUF + s) * BB, BB)],
                sems.at[s],
            ).wait()


def kernel(outputs, outputs_length):
    ids = outputs.reshape(BATCH, TIME, 1).astype(jnp.int32)
    out = pl.pallas_call(
        _one_hot_body,
        grid=(STEPS,),
        in_specs=[pl.BlockSpec((BB, TIME, 1), lambda i: (i, 0, 0))],
        out_specs=pl.BlockSpec(memory_space=pltpu.MemorySpace.HBM),
        out_shape=jax.ShapeDtypeStruct((BATCH, TIME, NUM_CLASSES), jnp.float32),
        scratch_shapes=[
            pltpu.VMEM((NBUF, BB, TIME, NUM_CLASSES), jnp.float32),
            pltpu.SemaphoreType.DMA((NBUF,)),
        ],
    )(ids)
    return out, outputs_length


# speedup vs baseline: 1.0301x; 1.0301x over previous
import jax
import jax.numpy as jnp
from jax import lax
from jax.experimental import pallas as pl
from jax.experimental.pallas import tpu as pltpu

BLANK_ID = 0
NUM_CLASSES = 1000
BATCH = 1024
TIME = 50
BB = 16
NBUF = 8
OUTER = BATCH // (BB * NBUF)


def _one_hot_body(ids_ref, out_ref, buf, sems):
    i = pl.program_id(0)
    iota = lax.broadcasted_iota(jnp.int32, (BB, TIME, NUM_CLASSES), 2)
    for s in range(NBUF):
        @pl.when(i >= 1)
        def _wait_reuse(s=s):
            pltpu.make_async_copy(
                buf.at[s],
                out_ref.at[pl.ds(((i - 1) * NBUF + s) * BB, BB)],
                sems.at[s],
            ).wait()

        ids = ids_ref[pl.ds((i * NBUF + s) * BB, BB)]
        sel = jnp.where(ids == BLANK_ID, -1, ids)
        buf[s] = (iota == sel).astype(jnp.float32)

        pltpu.make_async_copy(
            buf.at[s],
            out_ref.at[pl.ds((i * NBUF + s) * BB, BB)],
            sems.at[s],
        ).start(priority=s % 2)

    @pl.when(i == OUTER - 1)
    def _drain():
        for s in range(NBUF):
            pltpu.make_async_copy(
                buf.at[s],
                out_ref.at[pl.ds((i * NBUF + s) * BB, BB)],
                sems.at[s],
            ).wait()


def kernel(outputs, outputs_length):
    ids = outputs.reshape(BATCH, TIME, 1).astype(jnp.int32)
    out = pl.pallas_call(
        _one_hot_body,
        grid=(OUTER,),
        in_specs=[pl.BlockSpec((BATCH, TIME, 1), lambda i: (0, 0, 0))],
        out_specs=pl.BlockSpec(memory_space=pltpu.MemorySpace.HBM),
        out_shape=jax.ShapeDtypeStruct((BATCH, TIME, NUM_CLASSES), jnp.float32),
        scratch_shapes=[
            pltpu.VMEM((NBUF, BB, TIME, NUM_CLASSES), jnp.float32),
            pltpu.SemaphoreType.DMA((NBUF,)),
        ],
    )(ids)
    return out, outputs_length


# speedup vs baseline: 1.1045x; 1.0723x over previous
import jax
import jax.numpy as jnp
from jax import lax
from jax.experimental import pallas as pl
from jax.experimental.pallas import tpu as pltpu

BLANK_ID = 0
NUM_CLASSES = 1000
BATCH = 1024
TIME = 50
BB = 16
NBUF = 8
OUTER = BATCH // (BB * NBUF)


def _one_hot_body(ids_ref, out_ref, buf, sems):
    i = pl.program_id(0)
    iota = lax.broadcasted_iota(jnp.int32, (BB, TIME, NUM_CLASSES), 2)
    for s in range(NBUF):
        @pl.when(i >= 1)
        def _wait_reuse(s=s):
            pltpu.make_async_copy(
                buf.at[s],
                out_ref.at[pl.ds(((i - 1) * NBUF + s) * BB, BB)],
                sems.at[s],
            ).wait()

        ids = ids_ref[pl.ds((i * NBUF + s) * BB, BB)]
        sel = jnp.where(ids == BLANK_ID, -1, ids)[:, :, None]
        buf[s] = (iota == sel).astype(jnp.float32)

        pltpu.make_async_copy(
            buf.at[s],
            out_ref.at[pl.ds((i * NBUF + s) * BB, BB)],
            sems.at[s],
        ).start(priority=s % 2)

    @pl.when(i == OUTER - 1)
    def _drain():
        for s in range(NBUF):
            pltpu.make_async_copy(
                buf.at[s],
                out_ref.at[pl.ds((i * NBUF + s) * BB, BB)],
                sems.at[s],
            ).wait()


def kernel(outputs, outputs_length):
    ids = outputs.astype(jnp.int32)
    out = pl.pallas_call(
        _one_hot_body,
        grid=(OUTER,),
        in_specs=[pl.BlockSpec((BATCH, TIME), lambda i: (0, 0))],
        out_specs=pl.BlockSpec(memory_space=pltpu.MemorySpace.HBM),
        out_shape=jax.ShapeDtypeStruct((BATCH, TIME, NUM_CLASSES), jnp.float32),
        scratch_shapes=[
            pltpu.VMEM((NBUF, BB, TIME, NUM_CLASSES), jnp.float32),
            pltpu.SemaphoreType.DMA((NBUF,)),
        ],
    )(ids)
    return out, outputs_length
